# Initial kernel scaffold; baseline (speedup 1.0000x reference)
import jax
import jax.numpy as jnp
from jax.experimental import pallas as pl


def kernel(input, label):
    raise NotImplementedError("write your pallas kernel here")


# speedup vs baseline: 1.0412x; 1.0412x over previous
import jax
import jax.numpy as jnp
from jax.experimental import pallas as pl
from jax.experimental.pallas import tpu as pltpu

_S = 30.0
_M = 0.35


def _cosface_body(n_cols, n_blocks, bc, x_ref, lbl_ref, out_ref,
                  m_ref, s_ref, t_ref):
    i = pl.program_id(0)

    @pl.when(i == 0)
    def _init():
        m_ref[...] = jnp.full_like(m_ref, -jnp.inf)
        s_ref[...] = jnp.zeros_like(s_ref)
        t_ref[...] = jnp.zeros_like(t_ref)

    xb = x_ref[...]
    lbl = lbl_ref[...]
    colids = jax.lax.broadcasted_iota(jnp.int32, xb.shape, 1) + i * bc
    match = colids == lbl
    xv = jnp.where(colids < n_cols, xb, -jnp.inf)
    z = jnp.where(match, xv - _M, xv)

    m_old = m_ref[...]
    m_new = jnp.maximum(m_old, jnp.max(z, axis=1, keepdims=True))
    p = jnp.exp(_S * (z - m_new))
    s_ref[...] = s_ref[...] * jnp.exp(_S * (m_old - m_new)) \
        + jnp.sum(p, axis=1, keepdims=True)
    m_ref[...] = m_new
    t_ref[...] = t_ref[...] + jnp.sum(jnp.where(match, xv, 0.0),
                                      axis=1, keepdims=True)

    @pl.when(i == n_blocks - 1)
    def _fin():
        nll = jnp.log(s_ref[...]) + _S * m_ref[...] - _S * (t_ref[...] - _M)
        out_ref[...] = jnp.sum(nll, axis=(0, 1), keepdims=True) / nll.shape[0]


@jax.jit
def kernel(input, label):
    n_rows, n_cols = input.shape
    bc = 1024
    n_blocks = pl.cdiv(n_cols, bc)
    lbl = label.astype(jnp.int32).reshape(n_rows, 1)

    body = lambda *refs: _cosface_body(n_cols, n_blocks, bc, *refs)
    out = pl.pallas_call(
        body,
        grid=(n_blocks,),
        in_specs=[
            pl.BlockSpec((n_rows, bc), lambda i: (0, i)),
            pl.BlockSpec((n_rows, 1), lambda i: (0, 0)),
        ],
        out_specs=pl.BlockSpec((1, 1), lambda i: (0, 0)),
        out_shape=jax.ShapeDtypeStruct((1, 1), jnp.float32),
        scratch_shapes=[
            pltpu.VMEM((n_rows, 1), jnp.float32),
            pltpu.VMEM((n_rows, 1), jnp.float32),
            pltpu.VMEM((n_rows, 1), jnp.float32),
        ],
    )(input, lbl)
    return out[0, 0]
